# Initial kernel scaffold; baseline (speedup 1.0000x reference)
import jax
import jax.numpy as jnp
from jax.experimental import pallas as pl


def kernel(x, support_x, support_y, W_enc, b_enc, W_q, b_q, W_cls, b_cls, mem_keys, mem_values):
    raise NotImplementedError("write your pallas kernel here")


# speedup vs baseline: 3.1763x; 3.1763x over previous
import functools

import jax
import jax.numpy as jnp
from jax.experimental import pallas as pl
from jax.experimental.pallas import tpu as pltpu

HEADS = 8
LANE = 128


def _fused_kernel(x_ref, W_enc_ref, b_enc_ref, sx_ref, y_ref, Wq64_ref,
                  bq64_ref, Wc1_ref, Wc2_ref, bcls_ref, out_ref,
                  k64_ref, M_ref, *, head_dim, inv_sqrt_d):
    pid = pl.program_id(0)

    @pl.when(pid == 0)
    def _prologue():
        sf64 = jnp.dot(sx_ref[...], W_enc_ref[:, :head_dim],
                       preferred_element_type=jnp.float32)
        k64_ref[...] = sf64 + b_enc_ref[0, :head_dim]
        oh = (y_ref[...] == jax.lax.broadcasted_iota(
            jnp.int32, (y_ref.shape[0], 8), 1)).astype(jnp.float32)
        M_ref[...] = jnp.dot(oh, Wc2_ref[...],
                             preferred_element_type=jnp.float32)

    f = jnp.dot(x_ref[...], W_enc_ref[...],
                preferred_element_type=jnp.float32) + b_enc_ref[...]
    q64 = jnp.dot(f, Wq64_ref[...],
                  preferred_element_type=jnp.float32) + bq64_ref[...]
    s = jax.lax.dot_general(q64, k64_ref[...], (((1,), (1,)), ((), ())),
                            preferred_element_type=jnp.float32) * inv_sqrt_d
    m = jnp.max(s, axis=-1, keepdims=True)
    e = jnp.exp(s - m)
    p = e / jnp.sum(e, axis=-1, keepdims=True)
    out_ref[...] = (jnp.dot(f, Wc1_ref[...],
                            preferred_element_type=jnp.float32)
                    + jnp.dot(p, M_ref[...],
                              preferred_element_type=jnp.float32)
                    + bcls_ref[...])


def kernel(x, support_x, support_y, W_enc, b_enc, W_q, b_q, W_cls, b_cls,
           mem_keys, mem_values):
    B, DIN = x.shape
    FEAT = W_enc.shape[1]
    S = support_x.shape[0]
    NWAY = W_cls.shape[1]
    head_dim = FEAT // HEADS

    b_enc2 = b_enc.reshape(1, FEAT)
    Wq64 = W_q[:, :head_dim]
    bq64 = b_q[:head_dim].reshape(1, head_dim)
    Wc1p = jnp.pad(W_cls[:FEAT], ((0, 0), (0, LANE - NWAY)))
    Wc2p = jnp.pad(W_cls[FEAT:FEAT + NWAY], ((0, 8 - NWAY), (0, LANE - NWAY)))
    bclsp = jnp.pad(b_cls, (0, LANE - NWAY)).reshape(1, LANE)
    y2d = support_y.astype(jnp.int32).reshape(S, 1)

    BLK = 512
    grid = (B // BLK,)
    body = functools.partial(_fused_kernel, head_dim=head_dim,
                             inv_sqrt_d=float(1.0 / (head_dim ** 0.5)))
    out = pl.pallas_call(
        body,
        grid=grid,
        in_specs=[
            pl.BlockSpec((BLK, DIN), lambda i: (i, 0)),
            pl.BlockSpec((DIN, FEAT), lambda i: (0, 0)),
            pl.BlockSpec((1, FEAT), lambda i: (0, 0)),
            pl.BlockSpec((S, DIN), lambda i: (0, 0)),
            pl.BlockSpec((S, 1), lambda i: (0, 0)),
            pl.BlockSpec((FEAT, head_dim), lambda i: (0, 0)),
            pl.BlockSpec((1, head_dim), lambda i: (0, 0)),
            pl.BlockSpec((FEAT, LANE), lambda i: (0, 0)),
            pl.BlockSpec((8, LANE), lambda i: (0, 0)),
            pl.BlockSpec((1, LANE), lambda i: (0, 0)),
        ],
        out_specs=pl.BlockSpec((BLK, LANE), lambda i: (i, 0)),
        out_shape=jax.ShapeDtypeStruct((B, LANE), jnp.float32),
        scratch_shapes=[
            pltpu.VMEM((S, head_dim), jnp.float32),
            pltpu.VMEM((S, LANE), jnp.float32),
        ],
    )(x, W_enc, b_enc2, support_x, y2d, Wq64, bq64, Wc1p, Wc2p, bclsp)
    return out[:, :NWAY]
